# baseline (speedup 1.0000x reference)
import functools
import math

import jax
import jax.numpy as jnp
from jax import lax
from jax.experimental import pallas as pl
from jax.experimental.pallas import tpu as pltpu
from jax.experimental.pallas import tpu_sc as plsc

N_NODES = 10000
D = 128

NC = 2
NS = 16
NW = NC * NS

CHUNK = 128
BLKC = 16
ROWS_PER_SUB = 632
N_PAD = NS * ROWS_PER_SUB


@functools.lru_cache(maxsize=None)
def _sc_scatter(n_chunks):
  assert n_chunks % BLKC == 0
  nb = n_chunks // BLKC
  mesh = plsc.VectorSubcoreMesh(core_axis_name="c", subcore_axis_name="s")

  @functools.partial(
      pl.kernel,
      mesh=mesh,
      out_type=jax.ShapeDtypeStruct((NC, N_PAD, D), jnp.float32),
      scratch_types=[
          pltpu.VMEM((n_chunks, CHUNK), jnp.int32),
          pltpu.VMEM((n_chunks, CHUNK), jnp.int32),
          pltpu.VMEM((1, CHUNK, D), jnp.float32),
          pltpu.VMEM_SHARED((N_PAD, D), jnp.float32),
          pltpu.SemaphoreType.DMA,
      ],
  )
  def sc_scatter(x_hbm, src_hbm, dst_hbm, zeros_hbm, out_hbm,
                 src_v, dst_v, rows_v, acc_sh, sem0):
    c = lax.axis_index("c")
    s = lax.axis_index("s")
    wid = s * NC + c

    pltpu.sync_copy(zeros_hbm.at[pl.ds(s * ROWS_PER_SUB, ROWS_PER_SUB)],
                    acc_sh.at[pl.ds(s * ROWS_PER_SUB, ROWS_PER_SUB)])

    pltpu.sync_copy(src_hbm.at[wid], src_v)
    pltpu.sync_copy(dst_hbm.at[wid], dst_v)

    plsc.subcore_barrier()

    def seq_body(j, carry):
      pltpu.async_copy(x_hbm.at[src_v.at[j]], rows_v.at[0], sem0).wait()
      pltpu.sync_copy(rows_v.at[0], acc_sh.at[dst_v.at[j]], add=True)
      return carry

    lax.fori_loop(0, n_chunks, seq_body, 0)

    plsc.subcore_barrier()

    pltpu.sync_copy(acc_sh.at[pl.ds(s * ROWS_PER_SUB, ROWS_PER_SUB)],
                    out_hbm.at[c].at[pl.ds(s * ROWS_PER_SUB, ROWS_PER_SUB)])

  return sc_scatter


BLK = 1000


def _tc_body(p0_ref, p1_ref, w_ref, b_ref, o_ref):
  acc = p0_ref[...] + p1_ref[...]
  o_ref[...] = (
      jnp.dot(acc, w_ref[...], preferred_element_type=jnp.float32) + b_ref[...]
  )


def _tc_finish(p0, p1, W, b):
  grid = (N_NODES // BLK,)
  return pl.pallas_call(
      _tc_body,
      grid=grid,
      in_specs=[
          pl.BlockSpec((BLK, D), lambda i: (i, 0)),
          pl.BlockSpec((BLK, D), lambda i: (i, 0)),
          pl.BlockSpec((D, D), lambda i: (0, 0)),
          pl.BlockSpec((1, D), lambda i: (0, 0)),
      ],
      out_specs=pl.BlockSpec((BLK, D), lambda i: (i, 0)),
      out_shape=jax.ShapeDtypeStruct((N_NODES, D), jnp.float32),
  )(p0, p1, W, b.reshape(1, D))


def kernel(input, edge_index, W, b):
  dst = edge_index[0].astype(jnp.int32)
  src = edge_index[1].astype(jnp.int32)
  E = src.shape[0]
  per_blk = NW * CHUNK * BLKC
  n_chunks = BLKC * math.ceil(E / per_blk)
  e_pad = NW * n_chunks * CHUNK
  pad = e_pad - E
  if pad:
    src = jnp.concatenate([src, jnp.zeros((pad,), jnp.int32)])
    pad_dst = N_NODES + (jnp.arange(pad, dtype=jnp.int32) % (N_PAD - N_NODES))
    dst = jnp.concatenate([dst, pad_dst])
  src3 = src.reshape(NW, n_chunks, CHUNK)
  dst3 = dst.reshape(NW, n_chunks, CHUNK)
  zeros = jnp.zeros((N_PAD, D), jnp.float32)

  partials = _sc_scatter(n_chunks)(input, src3, dst3, zeros)
  p = partials[:, :N_NODES]
  return _tc_finish(p[0], p[1], W, b)


# speedup vs baseline: 2.8180x; 2.8180x over previous
import functools
import math

import jax
import jax.numpy as jnp
from jax import lax
from jax.experimental import pallas as pl
from jax.experimental.pallas import tpu as pltpu
from jax.experimental.pallas import tpu_sc as plsc

N_NODES = 10000
D = 128

NC = 2
NS = 16
NW = NC * NS

CHUNK = 128
BLKC = 16
ROWS_PER_SUB = 640
N_PAD = NS * ROWS_PER_SUB


@functools.lru_cache(maxsize=None)
def _sc_scatter(n_chunks):
  assert n_chunks % BLKC == 0
  nb = n_chunks // BLKC
  mesh = plsc.VectorSubcoreMesh(core_axis_name="c", subcore_axis_name="s")

  @functools.partial(
      pl.kernel,
      mesh=mesh,
      out_type=jax.ShapeDtypeStruct((NC, N_PAD, D), jnp.float32),
      scratch_types=[
          pltpu.VMEM((n_chunks, CHUNK), jnp.int32),
          pltpu.VMEM((n_chunks, CHUNK), jnp.int32),
          pltpu.VMEM((1, CHUNK, D), jnp.float32),
          pltpu.VMEM_SHARED((N_PAD, D), jnp.float32),
          pltpu.SemaphoreType.DMA,
      ],
  )
  def sc_scatter(x_hbm, src_hbm, dst_hbm, zeros_hbm, out_hbm,
                 src_v, dst_v, rows_v, acc_sh, sem0):
    c = lax.axis_index("c")
    s = lax.axis_index("s")
    wid = s * NC + c

    pltpu.sync_copy(zeros_hbm.at[pl.ds(s * ROWS_PER_SUB, ROWS_PER_SUB)],
                    acc_sh.at[pl.ds(s * ROWS_PER_SUB, ROWS_PER_SUB)])

    pltpu.sync_copy(src_hbm.at[wid], src_v)
    pltpu.sync_copy(dst_hbm.at[wid], dst_v)

    plsc.subcore_barrier()

    def seq_body(j, carry):
      pltpu.async_copy(x_hbm.at[src_v.at[j]], rows_v.at[0], sem0).wait()
      pltpu.sync_copy(rows_v.at[0], acc_sh.at[dst_v.at[j]], add=True)
      return carry

    lax.fori_loop(0, n_chunks, seq_body, 0)

    plsc.subcore_barrier()

    pltpu.sync_copy(acc_sh.at[pl.ds(s * ROWS_PER_SUB, ROWS_PER_SUB)],
                    out_hbm.at[c].at[pl.ds(s * ROWS_PER_SUB, ROWS_PER_SUB)])

  return sc_scatter


BLK = 1000


def _tc_body(p0_ref, p1_ref, w_ref, b_ref, o_ref):
  acc = p0_ref[...] + p1_ref[...]
  o_ref[...] = (
      jnp.dot(acc, w_ref[...], preferred_element_type=jnp.float32) + b_ref[...]
  )


def _tc_finish(p0, p1, W, b):
  grid = (N_NODES // BLK,)
  return pl.pallas_call(
      _tc_body,
      grid=grid,
      in_specs=[
          pl.BlockSpec((BLK, D), lambda i: (i, 0)),
          pl.BlockSpec((BLK, D), lambda i: (i, 0)),
          pl.BlockSpec((D, D), lambda i: (0, 0)),
          pl.BlockSpec((1, D), lambda i: (0, 0)),
      ],
      out_specs=pl.BlockSpec((BLK, D), lambda i: (i, 0)),
      out_shape=jax.ShapeDtypeStruct((N_NODES, D), jnp.float32),
  )(p0, p1, W, b.reshape(1, D))


def kernel(input, edge_index, W, b):
  dst = edge_index[0].astype(jnp.int32)
  src = edge_index[1].astype(jnp.int32)
  E = src.shape[0]
  per_blk = NW * CHUNK * BLKC
  n_chunks = BLKC * math.ceil(E / per_blk)
  e_pad = NW * n_chunks * CHUNK
  pad = e_pad - E
  if pad:
    ar = jnp.arange(pad, dtype=jnp.int32)
    src = jnp.concatenate([src, ar % N_NODES])
    dst = jnp.concatenate([dst, N_NODES + ar % (N_PAD - N_NODES)])
  src3 = src.reshape(n_chunks, NW, CHUNK).transpose(1, 0, 2)
  dst3 = dst.reshape(n_chunks, NW, CHUNK).transpose(1, 0, 2)
  zeros = jnp.zeros((N_PAD, D), jnp.float32)

  partials = _sc_scatter(n_chunks)(input, src3, dst3, zeros)
  p = partials[:, :N_NODES]
  return _tc_finish(p[0], p[1], W, b)


# speedup vs baseline: 3.7614x; 1.3347x over previous
import functools
import math

import jax
import jax.numpy as jnp
from jax import lax
from jax.experimental import pallas as pl
from jax.experimental.pallas import tpu as pltpu
from jax.experimental.pallas import tpu_sc as plsc

N_NODES = 10000
D = 128

NC = 2
NS = 16
NW = NC * NS

CHUNK = 128
BLKC = 16
ROWS_PER_SUB = 640
N_PAD = NS * ROWS_PER_SUB


@functools.lru_cache(maxsize=None)
def _sc_scatter(n_chunks):
  assert n_chunks % BLKC == 0
  nb = n_chunks // BLKC
  mesh = plsc.VectorSubcoreMesh(core_axis_name="c", subcore_axis_name="s")

  @functools.partial(
      pl.kernel,
      mesh=mesh,
      out_type=jax.ShapeDtypeStruct((NC, N_PAD, D), jnp.float32),
      scratch_types=[
          pltpu.VMEM((BLKC, CHUNK), jnp.int32),
          pltpu.VMEM((BLKC, CHUNK), jnp.int32),
          pltpu.VMEM((2, CHUNK, D), jnp.float32),
          pltpu.VMEM_SHARED((N_PAD, D), jnp.float32),
          pltpu.SemaphoreType.DMA,
          pltpu.SemaphoreType.DMA,
      ],
  )
  def sc_scatter(x_hbm, src_hbm, dst_hbm, zeros_hbm, out_hbm,
                 src_v, dst_v, rows_v, acc_sh, sem0, sem1):
    c = lax.axis_index("c")
    s = lax.axis_index("s")
    wid = s * NC + c

    pltpu.sync_copy(zeros_hbm.at[pl.ds(s * ROWS_PER_SUB, ROWS_PER_SUB)],
                    acc_sh.at[pl.ds(s * ROWS_PER_SUB, ROWS_PER_SUB)])

    plsc.subcore_barrier()

    sems = (sem0, sem1)

    def start_gather(k):
      return pltpu.async_copy(x_hbm.at[src_v.at[k]], rows_v.at[k % 2],
                              sems[k % 2])

    def scatter(k):
      pltpu.sync_copy(rows_v.at[k % 2], acc_sh.at[dst_v.at[k]], add=True)

    def block_body(b, carry):
      pltpu.sync_copy(src_hbm.at[wid].at[pl.ds(b * BLKC, BLKC)], src_v)
      pltpu.sync_copy(dst_hbm.at[wid].at[pl.ds(b * BLKC, BLKC)], dst_v)
      g = [None, None]
      g[0] = start_gather(0)
      for k in range(1, BLKC + 1):
        if k < BLKC:
          g[k % 2] = start_gather(k)
        g[(k - 1) % 2].wait()
        scatter(k - 1)
      return carry

    lax.fori_loop(0, nb, block_body, 0)

    plsc.subcore_barrier()

    pltpu.sync_copy(acc_sh.at[pl.ds(s * ROWS_PER_SUB, ROWS_PER_SUB)],
                    out_hbm.at[c].at[pl.ds(s * ROWS_PER_SUB, ROWS_PER_SUB)])

  return sc_scatter


BLK = 1000


def _tc_body(p0_ref, p1_ref, w_ref, b_ref, o_ref):
  acc = p0_ref[...] + p1_ref[...]
  o_ref[...] = (
      jnp.dot(acc, w_ref[...], preferred_element_type=jnp.float32) + b_ref[...]
  )


def _tc_finish(p0, p1, W, b):
  grid = (N_NODES // BLK,)
  return pl.pallas_call(
      _tc_body,
      grid=grid,
      in_specs=[
          pl.BlockSpec((BLK, D), lambda i: (i, 0)),
          pl.BlockSpec((BLK, D), lambda i: (i, 0)),
          pl.BlockSpec((D, D), lambda i: (0, 0)),
          pl.BlockSpec((1, D), lambda i: (0, 0)),
      ],
      out_specs=pl.BlockSpec((BLK, D), lambda i: (i, 0)),
      out_shape=jax.ShapeDtypeStruct((N_NODES, D), jnp.float32),
  )(p0, p1, W, b.reshape(1, D))


def kernel(input, edge_index, W, b):
  dst = edge_index[0].astype(jnp.int32)
  src = edge_index[1].astype(jnp.int32)
  E = src.shape[0]
  per_blk = NW * CHUNK * BLKC
  n_chunks = BLKC * math.ceil(E / per_blk)
  e_pad = NW * n_chunks * CHUNK
  pad = e_pad - E
  if pad:
    ar = jnp.arange(pad, dtype=jnp.int32)
    src = jnp.concatenate([src, ar % N_NODES])
    dst = jnp.concatenate([dst, N_NODES + ar % (N_PAD - N_NODES)])
  src3 = src.reshape(n_chunks, NW, CHUNK).transpose(1, 0, 2)
  dst3 = dst.reshape(n_chunks, NW, CHUNK).transpose(1, 0, 2)
  zeros = jnp.zeros((N_PAD, D), jnp.float32)

  partials = _sc_scatter(n_chunks)(input, src3, dst3, zeros)
  p = partials[:, :N_NODES]
  return _tc_finish(p[0], p[1], W, b)


# speedup vs baseline: 3.9767x; 1.0572x over previous
import functools
import math

import jax
import jax.numpy as jnp
from jax import lax
from jax.experimental import pallas as pl
from jax.experimental.pallas import tpu as pltpu
from jax.experimental.pallas import tpu_sc as plsc

N_NODES = 10000
D = 128

NC = 2
NS = 16
NW = NC * NS

CHUNK = 128
BLKC = 16
ROWS_PER_SUB = 640
N_PAD = NS * ROWS_PER_SUB


@functools.lru_cache(maxsize=None)
def _sc_scatter(n_chunks):
  assert n_chunks % BLKC == 0
  nb = n_chunks // BLKC
  mesh = plsc.VectorSubcoreMesh(core_axis_name="c", subcore_axis_name="s")

  @functools.partial(
      pl.kernel,
      mesh=mesh,
      out_type=jax.ShapeDtypeStruct((NC, N_PAD, D), jnp.float32),
      scratch_types=[
          pltpu.VMEM((2, BLKC, CHUNK), jnp.int32),
          pltpu.VMEM((2, BLKC, CHUNK), jnp.int32),
          pltpu.VMEM((2, CHUNK, D), jnp.float32),
          pltpu.VMEM_SHARED((N_PAD, D), jnp.float32),
          pltpu.SemaphoreType.DMA,
          pltpu.SemaphoreType.DMA,
          pltpu.SemaphoreType.DMA,
      ],
  )
  def sc_scatter(x_hbm, src_hbm, dst_hbm, zeros_hbm, out_hbm,
                 src_v, dst_v, rows_v, acc_sh, sem0, sem1, sem_idx):
    c = lax.axis_index("c")
    s = lax.axis_index("s")
    wid = s * NC + c

    pltpu.sync_copy(zeros_hbm.at[pl.ds(s * ROWS_PER_SUB, ROWS_PER_SUB)],
                    acc_sh.at[pl.ds(s * ROWS_PER_SUB, ROWS_PER_SUB)])

    pltpu.sync_copy(src_hbm.at[wid].at[pl.ds(0, BLKC)], src_v.at[0])
    pltpu.sync_copy(dst_hbm.at[wid].at[pl.ds(0, BLKC)], dst_v.at[0])

    plsc.subcore_barrier()

    sems = (sem0, sem1)

    def stage_block(b):
      bb = b % 2
      h0 = pltpu.async_copy(src_hbm.at[wid].at[pl.ds(b * BLKC, BLKC)],
                            src_v.at[bb], sem_idx)
      h1 = pltpu.async_copy(dst_hbm.at[wid].at[pl.ds(b * BLKC, BLKC)],
                            dst_v.at[bb], sem_idx)
      return (h0, h1)

    def start_gather(j):
      b, k = j // BLKC, j % BLKC
      return pltpu.async_copy(x_hbm.at[src_v.at[b % 2].at[k]],
                              rows_v.at[j % 2], sems[j % 2])

    def scatter(j):
      b, k = j // BLKC, j % BLKC
      pltpu.sync_copy(rows_v.at[j % 2], acc_sh.at[dst_v.at[b % 2].at[k]],
                      add=True)

    idx_pending = stage_block(1) if nb > 1 else None
    g = [None, None]
    g[0] = start_gather(0)
    for j in range(1, n_chunks + 1):
      if j < n_chunks:
        if j % BLKC == 0:
          for h in idx_pending:
            h.wait()
        g[j % 2] = start_gather(j)
      g[(j - 1) % 2].wait()
      scatter(j - 1)
      if j < n_chunks and j % BLKC == 0:
        b = j // BLKC
        idx_pending = stage_block(b + 1) if b + 1 < nb else None

    plsc.subcore_barrier()

    pltpu.sync_copy(acc_sh.at[pl.ds(s * ROWS_PER_SUB, ROWS_PER_SUB)],
                    out_hbm.at[c].at[pl.ds(s * ROWS_PER_SUB, ROWS_PER_SUB)])

  return sc_scatter


BLK = 1000


def _tc_body(p0_ref, p1_ref, w_ref, b_ref, o_ref):
  acc = p0_ref[...] + p1_ref[...]
  o_ref[...] = (
      jnp.dot(acc, w_ref[...], preferred_element_type=jnp.float32) + b_ref[...]
  )


def _tc_finish(p0, p1, W, b):
  grid = (N_NODES // BLK,)
  return pl.pallas_call(
      _tc_body,
      grid=grid,
      in_specs=[
          pl.BlockSpec((BLK, D), lambda i: (i, 0)),
          pl.BlockSpec((BLK, D), lambda i: (i, 0)),
          pl.BlockSpec((D, D), lambda i: (0, 0)),
          pl.BlockSpec((1, D), lambda i: (0, 0)),
      ],
      out_specs=pl.BlockSpec((BLK, D), lambda i: (i, 0)),
      out_shape=jax.ShapeDtypeStruct((N_NODES, D), jnp.float32),
  )(p0, p1, W, b.reshape(1, D))


def kernel(input, edge_index, W, b):
  dst = edge_index[0].astype(jnp.int32)
  src = edge_index[1].astype(jnp.int32)
  E = src.shape[0]
  per_blk = NW * CHUNK * BLKC
  n_chunks = BLKC * math.ceil(E / per_blk)
  e_pad = NW * n_chunks * CHUNK
  pad = e_pad - E
  if pad:
    ar = jnp.arange(pad, dtype=jnp.int32)
    src = jnp.concatenate([src, ar % N_NODES])
    dst = jnp.concatenate([dst, N_NODES + ar % (N_PAD - N_NODES)])
  src3 = src.reshape(n_chunks, NW, CHUNK).transpose(1, 0, 2)
  dst3 = dst.reshape(n_chunks, NW, CHUNK).transpose(1, 0, 2)
  zeros = jnp.zeros((N_PAD, D), jnp.float32)

  partials = _sc_scatter(n_chunks)(input, src3, dst3, zeros)
  p = partials[:, :N_NODES]
  return _tc_finish(p[0], p[1], W, b)


# speedup vs baseline: 3.9999x; 1.0058x over previous
import functools
import math

import jax
import jax.numpy as jnp
from jax import lax
from jax.experimental import pallas as pl
from jax.experimental.pallas import tpu as pltpu
from jax.experimental.pallas import tpu_sc as plsc

N_NODES = 10000
D = 128

NC = 2
NS = 16
NW = NC * NS

CHUNK = 128
BLKC = 16
ROWS_PER_SUB = 640
N_PAD = NS * ROWS_PER_SUB


@functools.lru_cache(maxsize=None)
def _sc_scatter(n_chunks):
  assert n_chunks % BLKC == 0
  nb = n_chunks // BLKC
  mesh = plsc.VectorSubcoreMesh(core_axis_name="c", subcore_axis_name="s")

  @functools.partial(
      pl.kernel,
      mesh=mesh,
      out_type=jax.ShapeDtypeStruct((NC, N_PAD, D), jnp.float32),
      scratch_types=[
          pltpu.VMEM((2, BLKC, CHUNK), jnp.int32),
          pltpu.VMEM((2, BLKC, CHUNK), jnp.int32),
          pltpu.VMEM((2, CHUNK, D), jnp.float32),
          pltpu.VMEM_SHARED((N_PAD, D), jnp.float32),
          pltpu.SemaphoreType.DMA,
          pltpu.SemaphoreType.DMA,
          pltpu.SemaphoreType.DMA,
      ],
  )
  def sc_scatter(x_hbm, src_hbm, dst_hbm, zeros_hbm, out_hbm,
                 src_v, dst_v, rows_v, acc_sh, sem0, sem1, sem_idx):
    c = lax.axis_index("c")
    s = lax.axis_index("s")
    wid = s * NC + c

    pltpu.sync_copy(zeros_hbm.at[pl.ds(s * ROWS_PER_SUB, ROWS_PER_SUB)],
                    acc_sh.at[pl.ds(s * ROWS_PER_SUB, ROWS_PER_SUB)])

    col = wid * CHUNK

    pltpu.sync_copy(src_hbm.at[pl.ds(0, BLKC), pl.ds(col, CHUNK)], src_v.at[0])
    pltpu.sync_copy(dst_hbm.at[pl.ds(0, BLKC), pl.ds(col, CHUNK)], dst_v.at[0])

    plsc.subcore_barrier()

    sems = (sem0, sem1)

    def stage_block(b):
      bb = b % 2
      h0 = pltpu.async_copy(
          src_hbm.at[pl.ds(b * BLKC, BLKC), pl.ds(col, CHUNK)],
          src_v.at[bb], sem_idx)
      h1 = pltpu.async_copy(
          dst_hbm.at[pl.ds(b * BLKC, BLKC), pl.ds(col, CHUNK)],
          dst_v.at[bb], sem_idx)
      return (h0, h1)

    def start_gather(j):
      b, k = j // BLKC, j % BLKC
      return pltpu.async_copy(x_hbm.at[src_v.at[b % 2].at[k]],
                              rows_v.at[j % 2], sems[j % 2])

    def scatter(j):
      b, k = j // BLKC, j % BLKC
      pltpu.sync_copy(rows_v.at[j % 2], acc_sh.at[dst_v.at[b % 2].at[k]],
                      add=True)

    idx_pending = stage_block(1) if nb > 1 else None
    g = [None, None]
    g[0] = start_gather(0)
    for j in range(1, n_chunks + 1):
      if j < n_chunks:
        if j % BLKC == 0:
          for h in idx_pending:
            h.wait()
        g[j % 2] = start_gather(j)
      g[(j - 1) % 2].wait()
      scatter(j - 1)
      if j < n_chunks and j % BLKC == 0:
        b = j // BLKC
        idx_pending = stage_block(b + 1) if b + 1 < nb else None

    plsc.subcore_barrier()

    pltpu.sync_copy(acc_sh.at[pl.ds(s * ROWS_PER_SUB, ROWS_PER_SUB)],
                    out_hbm.at[c].at[pl.ds(s * ROWS_PER_SUB, ROWS_PER_SUB)])

  return sc_scatter


BLK = 1000


def _tc_body(p0_ref, p1_ref, w_ref, b_ref, o_ref):
  acc = p0_ref[...] + p1_ref[...]
  o_ref[...] = (
      jnp.dot(acc, w_ref[...], preferred_element_type=jnp.float32) + b_ref[...]
  )


def _tc_finish(p0, p1, W, b):
  grid = (N_NODES // BLK,)
  return pl.pallas_call(
      _tc_body,
      grid=grid,
      in_specs=[
          pl.BlockSpec((BLK, D), lambda i: (i, 0)),
          pl.BlockSpec((BLK, D), lambda i: (i, 0)),
          pl.BlockSpec((D, D), lambda i: (0, 0)),
          pl.BlockSpec((1, D), lambda i: (0, 0)),
      ],
      out_specs=pl.BlockSpec((BLK, D), lambda i: (i, 0)),
      out_shape=jax.ShapeDtypeStruct((N_NODES, D), jnp.float32),
  )(p0, p1, W, b.reshape(1, D))


def kernel(input, edge_index, W, b):
  dst = edge_index[0].astype(jnp.int32)
  src = edge_index[1].astype(jnp.int32)
  E = src.shape[0]
  per_blk = NW * CHUNK * BLKC
  n_chunks = BLKC * math.ceil(E / per_blk)
  e_pad = NW * n_chunks * CHUNK
  pad = e_pad - E
  if pad:
    ar = jnp.arange(pad, dtype=jnp.int32)
    src = jnp.concatenate([src, ar % N_NODES])
    dst = jnp.concatenate([dst, N_NODES + ar % (N_PAD - N_NODES)])
  src3 = src.reshape(n_chunks, NW * CHUNK)
  dst3 = dst.reshape(n_chunks, NW * CHUNK)
  zeros = jnp.zeros((N_PAD, D), jnp.float32)

  partials = _sc_scatter(n_chunks)(input, src3, dst3, zeros)
  p = partials[:, :N_NODES]
  return _tc_finish(p[0], p[1], W, b)


# speedup vs baseline: 4.1680x; 1.0420x over previous
import functools
import math

import jax
import jax.numpy as jnp
from jax import lax
from jax.experimental import pallas as pl
from jax.experimental.pallas import tpu as pltpu
from jax.experimental.pallas import tpu_sc as plsc

N_NODES = 10000
D = 128

NC = 2
NS = 16
NW = NC * NS

CHUNK = 128
BLKC = 16
ROWS_PER_SUB = 640
N_PAD = NS * ROWS_PER_SUB


@functools.lru_cache(maxsize=None)
def _sc_scatter(n_chunks):
  assert n_chunks % BLKC == 0
  nb = n_chunks // BLKC
  mesh = plsc.VectorSubcoreMesh(core_axis_name="c", subcore_axis_name="s")

  @functools.partial(
      pl.kernel,
      mesh=mesh,
      out_type=jax.ShapeDtypeStruct((NC, N_PAD, D), jnp.float32),
      scratch_types=[
          pltpu.VMEM((2, BLKC, CHUNK), jnp.int32),
          pltpu.VMEM((2, BLKC, CHUNK), jnp.int32),
          pltpu.VMEM((2, CHUNK, D), jnp.float32),
          pltpu.VMEM_SHARED((N_PAD, D), jnp.float32),
          pltpu.SemaphoreType.DMA,
          pltpu.SemaphoreType.DMA,
          pltpu.SemaphoreType.DMA,
      ],
  )
  def sc_scatter(x_hbm, src_hbm, dst_hbm, out_hbm,
                 src_v, dst_v, rows_v, acc_sh, sem0, sem1, sem_idx):
    c = lax.axis_index("c")
    s = lax.axis_index("s")
    wid = s * NC + c

    zv = jnp.zeros((16,), jnp.float32)

    def zero_row(r, carry):
      for c8 in range(D // 16):
        rows_v[0, r, pl.ds(c8 * 16, 16)] = zv
      return carry

    lax.fori_loop(0, CHUNK, zero_row, 0)
    for rep in range(ROWS_PER_SUB // CHUNK):
      pltpu.sync_copy(
          rows_v.at[0],
          acc_sh.at[pl.ds(s * ROWS_PER_SUB + rep * CHUNK, CHUNK)])

    col = wid * CHUNK

    pltpu.sync_copy(src_hbm.at[pl.ds(0, BLKC), pl.ds(col, CHUNK)], src_v.at[0])
    pltpu.sync_copy(dst_hbm.at[pl.ds(0, BLKC), pl.ds(col, CHUNK)], dst_v.at[0])

    plsc.subcore_barrier()

    sems = (sem0, sem1)

    def stage_block(b):
      bb = b % 2
      h0 = pltpu.async_copy(
          src_hbm.at[pl.ds(b * BLKC, BLKC), pl.ds(col, CHUNK)],
          src_v.at[bb], sem_idx)
      h1 = pltpu.async_copy(
          dst_hbm.at[pl.ds(b * BLKC, BLKC), pl.ds(col, CHUNK)],
          dst_v.at[bb], sem_idx)
      return (h0, h1)

    def start_gather(j):
      b, k = j // BLKC, j % BLKC
      return pltpu.async_copy(x_hbm.at[src_v.at[b % 2].at[k]],
                              rows_v.at[j % 2], sems[j % 2])

    def scatter(j):
      b, k = j // BLKC, j % BLKC
      pltpu.sync_copy(rows_v.at[j % 2], acc_sh.at[dst_v.at[b % 2].at[k]],
                      add=True)

    idx_pending = stage_block(1) if nb > 1 else None
    g = [None, None]
    g[0] = start_gather(0)
    for j in range(1, n_chunks + 1):
      if j < n_chunks:
        if j % BLKC == 0:
          for h in idx_pending:
            h.wait()
        g[j % 2] = start_gather(j)
      g[(j - 1) % 2].wait()
      scatter(j - 1)
      if j < n_chunks and j % BLKC == 0:
        b = j // BLKC
        idx_pending = stage_block(b + 1) if b + 1 < nb else None

    plsc.subcore_barrier()

    pltpu.sync_copy(acc_sh.at[pl.ds(s * ROWS_PER_SUB, ROWS_PER_SUB)],
                    out_hbm.at[c].at[pl.ds(s * ROWS_PER_SUB, ROWS_PER_SUB)])

  return sc_scatter


BLK = 1000


def _tc_body(p0_ref, p1_ref, w_ref, b_ref, o_ref):
  acc = p0_ref[...] + p1_ref[...]
  o_ref[...] = (
      jnp.dot(acc, w_ref[...], preferred_element_type=jnp.float32) + b_ref[...]
  )


def _tc_finish(p0, p1, W, b):
  grid = (N_NODES // BLK,)
  return pl.pallas_call(
      _tc_body,
      grid=grid,
      in_specs=[
          pl.BlockSpec((BLK, D), lambda i: (i, 0)),
          pl.BlockSpec((BLK, D), lambda i: (i, 0)),
          pl.BlockSpec((D, D), lambda i: (0, 0)),
          pl.BlockSpec((1, D), lambda i: (0, 0)),
      ],
      out_specs=pl.BlockSpec((BLK, D), lambda i: (i, 0)),
      out_shape=jax.ShapeDtypeStruct((N_NODES, D), jnp.float32),
  )(p0, p1, W, b.reshape(1, D))


def kernel(input, edge_index, W, b):
  dst = edge_index[0].astype(jnp.int32)
  src = edge_index[1].astype(jnp.int32)
  E = src.shape[0]
  per_blk = NW * CHUNK * BLKC
  n_chunks = BLKC * math.ceil(E / per_blk)
  e_pad = NW * n_chunks * CHUNK
  pad = e_pad - E
  if pad:
    ar = jnp.arange(pad, dtype=jnp.int32)
    src = jnp.concatenate([src, ar % N_NODES])
    dst = jnp.concatenate([dst, N_NODES + ar % (N_PAD - N_NODES)])
  src3 = src.reshape(n_chunks, NW * CHUNK)
  dst3 = dst.reshape(n_chunks, NW * CHUNK)

  partials = _sc_scatter(n_chunks)(input, src3, dst3)
  p = partials[:, :N_NODES]
  return _tc_finish(p[0], p[1], W, b)


# speedup vs baseline: 4.4317x; 1.0633x over previous
import functools
import math

import jax
import jax.numpy as jnp
from jax import lax
from jax.experimental import pallas as pl
from jax.experimental.pallas import tpu as pltpu
from jax.experimental.pallas import tpu_sc as plsc

N_NODES = 10000
D = 128

NC = 2
NS = 16
NW = NC * NS

CHUNK = 128
BLKC = 16
ROWS_PER_SUB = 640
N_PAD = NS * ROWS_PER_SUB


@functools.lru_cache(maxsize=None)
def _sc_scatter(n_chunks):
  assert n_chunks % BLKC == 0
  nb = n_chunks // BLKC
  mesh = plsc.VectorSubcoreMesh(core_axis_name="c", subcore_axis_name="s")

  @functools.partial(
      pl.kernel,
      mesh=mesh,
      out_type=jax.ShapeDtypeStruct((NC, N_PAD, D), jnp.float32),
      scratch_types=[
          pltpu.VMEM((2, BLKC, CHUNK), jnp.int32),
          pltpu.VMEM((2, BLKC, CHUNK), jnp.int32),
          pltpu.VMEM((2, CHUNK, D), jnp.float32),
          pltpu.VMEM_SHARED((N_PAD, D), jnp.float32),
          pltpu.SemaphoreType.DMA,
          pltpu.SemaphoreType.DMA,
          pltpu.SemaphoreType.DMA,
      ],
  )
  def sc_scatter(x_hbm, src_hbm, dst_hbm, out_hbm,
                 src_v, dst_v, rows_v, acc_sh, sem0, sem1, sem_idx):
    c = lax.axis_index("c")
    s = lax.axis_index("s")
    wid = s * NC + c

    zv = jnp.zeros((16,), jnp.float32)

    def zero_row(r, carry):
      for c8 in range(D // 16):
        rows_v[0, r, pl.ds(c8 * 16, 16)] = zv
      return carry

    lax.fori_loop(0, CHUNK, zero_row, 0)
    for rep in range(ROWS_PER_SUB // CHUNK):
      pltpu.sync_copy(
          rows_v.at[0],
          acc_sh.at[pl.ds(s * ROWS_PER_SUB + rep * CHUNK, CHUNK)])

    col = wid * CHUNK

    pltpu.sync_copy(src_hbm.at[pl.ds(0, BLKC), pl.ds(col, CHUNK)], src_v.at[0])
    pltpu.sync_copy(dst_hbm.at[pl.ds(0, BLKC), pl.ds(col, CHUNK)], dst_v.at[0])

    plsc.subcore_barrier()

    sems = (sem0, sem1)

    def stage_block(b):
      bb = b % 2
      h0 = pltpu.async_copy(
          src_hbm.at[pl.ds(b * BLKC, BLKC), pl.ds(col, CHUNK)],
          src_v.at[bb], sem_idx)
      h1 = pltpu.async_copy(
          dst_hbm.at[pl.ds(b * BLKC, BLKC), pl.ds(col, CHUNK)],
          dst_v.at[bb], sem_idx)
      return (h0, h1)

    def start_gather(j):
      b, k = j // BLKC, j % BLKC
      return pltpu.async_copy(x_hbm.at[src_v.at[b % 2].at[k]],
                              rows_v.at[j % 2], sems[j % 2])

    def scatter(j):
      b, k = j // BLKC, j % BLKC
      pltpu.sync_copy(rows_v.at[j % 2], acc_sh.at[dst_v.at[b % 2].at[k]],
                      add=True)

    idx_pending = stage_block(1) if nb > 1 else None
    g = [None, None]
    g[0] = start_gather(0)
    for j in range(1, n_chunks + 1):
      if j < n_chunks:
        if j % BLKC == 0:
          for h in idx_pending:
            h.wait()
        g[j % 2] = start_gather(j)
      g[(j - 1) % 2].wait()
      scatter(j - 1)
      if j < n_chunks and j % BLKC == 0:
        b = j // BLKC
        idx_pending = stage_block(b + 1) if b + 1 < nb else None

    plsc.subcore_barrier()

    pltpu.sync_copy(acc_sh.at[pl.ds(s * ROWS_PER_SUB, ROWS_PER_SUB)],
                    out_hbm.at[c].at[pl.ds(s * ROWS_PER_SUB, ROWS_PER_SUB)])

  return sc_scatter


BLK = 2000


def _tc_body(p0_ref, p1_ref, w_ref, b_ref, o_ref):
  acc = p0_ref[0] + p1_ref[0]
  o_ref[...] = (
      jnp.dot(acc, w_ref[...], preferred_element_type=jnp.float32) + b_ref[...]
  )


def _tc_finish(partials, W, b):
  grid = (N_NODES // BLK,)
  return pl.pallas_call(
      _tc_body,
      grid=grid,
      in_specs=[
          pl.BlockSpec((1, BLK, D), lambda i: (0, i, 0)),
          pl.BlockSpec((1, BLK, D), lambda i: (1, i, 0)),
          pl.BlockSpec((D, D), lambda i: (0, 0)),
          pl.BlockSpec((1, D), lambda i: (0, 0)),
      ],
      out_specs=pl.BlockSpec((BLK, D), lambda i: (i, 0)),
      out_shape=jax.ShapeDtypeStruct((N_NODES, D), jnp.float32),
  )(partials, partials, W, b.reshape(1, D))


def kernel(input, edge_index, W, b):
  dst = edge_index[0].astype(jnp.int32)
  src = edge_index[1].astype(jnp.int32)
  E = src.shape[0]
  per_blk = NW * CHUNK * BLKC
  n_chunks = BLKC * math.ceil(E / per_blk)
  e_pad = NW * n_chunks * CHUNK
  pad = e_pad - E
  if pad:
    ar = jnp.arange(pad, dtype=jnp.int32)
    src = jnp.concatenate([src, ar % N_NODES])
    dst = jnp.concatenate([dst, N_NODES + ar % (N_PAD - N_NODES)])
  src3 = src.reshape(n_chunks, NW * CHUNK)
  dst3 = dst.reshape(n_chunks, NW * CHUNK)

  partials = _sc_scatter(n_chunks)(input, src3, dst3)
  return _tc_finish(partials, W, b)
